# Initial kernel scaffold; baseline (speedup 1.0000x reference)
import jax
import jax.numpy as jnp
from jax.experimental import pallas as pl


_BLOCK = 256


def _pe_kernel(table_ref, out_ref):
    out_ref[...] = jnp.broadcast_to(table_ref[None, ...], out_ref.shape)


def kernel(x, pos_table):
    batch, seq, _ = x.shape
    d = pos_table.shape[1]
    grid = (seq // _BLOCK,)
    return pl.pallas_call(
        _pe_kernel,
        grid=grid,
        in_specs=[pl.BlockSpec((_BLOCK, d), lambda j: (j, 0))],
        out_specs=pl.BlockSpec((batch, _BLOCK, d), lambda j: (0, j, 0)),
        out_shape=jax.ShapeDtypeStruct((batch, seq, d), pos_table.dtype),
    )(pos_table)


# speedup vs baseline: 4.7599x; 4.7599x over previous
import jax
import jax.numpy as jnp
from jax.experimental import pallas as pl


_BLOCK = 256


def _pe_kernel(table_ref, out_ref):
    tab = table_ref[...]
    out_ref[...] = jnp.broadcast_to(tab[None, :, :], out_ref.shape)


def kernel(x, pos_table):
    batch, seq, _ = x.shape
    d = pos_table.shape[1]
    grid = (seq // _BLOCK,)
    return pl.pallas_call(
        _pe_kernel,
        grid=grid,
        in_specs=[pl.BlockSpec((_BLOCK, d), lambda j: (j, 0))],
        out_specs=pl.BlockSpec((batch, _BLOCK, d), lambda j: (0, j, 0)),
        out_shape=jax.ShapeDtypeStruct((batch, seq, d), pos_table.dtype),
    )(pos_table)
